# baseline (speedup 1.0000x reference)
import jax
import jax.numpy as jnp
import numpy as np
from jax.experimental import pallas as pl
from jax.experimental.pallas import tpu as pltpu

T = 2048
HID = 2048
H = 16
DN = 128
DR = 64
DQK = DN + DR
DV = 128
RQ = 1536
RKV = 512
HI = 8
DI = 128
TOPK = 512
EPS = 1e-6
NEG = -1e9

BT = 256
NB = T // BT


def _dot(a, b, trans_b=False):
    dn = (((1,), (1,)), ((), ())) if trans_b else (((1,), (0,)), ((), ()))
    return jax.lax.dot_general(a, b, dn,
                               preferred_element_type=jnp.float32)


def _roll_lanes(x, shift):
    if shift > 0:
        return jnp.concatenate([x[:, -shift:], x[:, :-shift]], axis=1)
    k = -shift
    return jnp.concatenate([x[:, k:], x[:, :k]], axis=1)


def _rope_interleaved(x, c, s):
    lane = jax.lax.broadcasted_iota(jnp.int32, x.shape, 1)
    swap = jnp.where(lane % 2 == 0, _roll_lanes(x, -1), _roll_lanes(x, 1))
    return x * c + swap * s


def _rope_neox128(x, c, s):
    lane = jax.lax.broadcasted_iota(jnp.int32, x.shape, 1)
    swap = jnp.where(lane < 96, _roll_lanes(x, -32), _roll_lanes(x, 32))
    return x * c + swap * s


def _proj_q_kernel(qc_ref, wq_ref, wiq_ref, gw_ref, cil_ref, sil_ref,
                   cnx_ref, snx_ref, qn_ref, qpe_ref, qi_ref):
    x = qc_ref[...]
    xn = x * jax.lax.rsqrt(jnp.mean(x * x, axis=-1, keepdims=True) + EPS)
    xn = xn * gw_ref[...]
    q = _dot(xn, wq_ref[...])
    cil, sil = cil_ref[...], sil_ref[...]
    for h in range(H):
        base = h * DQK
        qn_ref[h] = q[:, base:base + DN].astype(jnp.bfloat16)
        pe = q[:, base + DN:base + DQK]
        qpe_ref[h] = _rope_interleaved(pe, cil, sil).astype(jnp.bfloat16)
    qi = _dot(xn, wiq_ref[...])
    cnx, snx = cnx_ref[...], snx_ref[...]
    for h in range(HI):
        g = qi[:, h * DI:(h + 1) * DI]
        qi_ref[h] = _rope_neox128(g, cnx, snx).astype(jnp.bfloat16)


def _proj_kv_kernel(kvc_ref, hid_ref, kpe_ref, wkv_ref, wik_ref, ww_ref,
                    gkv_ref, ilw_ref, ilb_ref, cil_ref, sil_ref,
                    cnx_ref, snx_ref, kn_ref, v_ref, kper_ref, ki_ref, wt_ref):
    x = kvc_ref[...]
    xn = x * jax.lax.rsqrt(jnp.mean(x * x, axis=-1, keepdims=True) + EPS)
    xn = xn * gkv_ref[...]
    kvb = _dot(xn, wkv_ref[...])
    kn_ref[...] = jnp.concatenate(
        [kvb[:, h * (DN + DV):h * (DN + DV) + DN] for h in range(H)],
        axis=1).astype(jnp.bfloat16)
    v_ref[...] = jnp.concatenate(
        [kvb[:, h * (DN + DV) + DN:(h + 1) * (DN + DV)] for h in range(H)],
        axis=1).astype(jnp.bfloat16)

    kper_ref[...] = _rope_interleaved(
        kpe_ref[...], cil_ref[...], sil_ref[...]).astype(jnp.bfloat16)

    hdd = hid_ref[...]
    ki0 = _dot(hdd, wik_ref[...])
    m = jnp.mean(ki0, axis=-1, keepdims=True)
    d = ki0 - m
    var = jnp.mean(d * d, axis=-1, keepdims=True)
    ki = d * jax.lax.rsqrt(var + 1e-6) * ilw_ref[...] + ilb_ref[...]
    ki_ref[...] = _rope_neox128(ki, cnx_ref[...], snx_ref[...]).astype(
        jnp.bfloat16)

    wt_ref[...] = _dot(hdd, ww_ref[...]) * (HI ** -0.5)


def _iscore_kernel(qi_ref, ki_ref, wt_ref, out_ref):
    i = pl.program_id(0)
    ki = ki_ref[...]
    scale = DI ** -0.5
    wtb = wt_ref[...].astype(jnp.bfloat16).astype(jnp.float32)
    acc = jnp.zeros((BT, T), jnp.float32)
    for h in range(HI):
        sh = _dot(qi_ref[h], ki, trans_b=True) * scale
        rb = jnp.maximum(sh, 0.0).astype(jnp.bfloat16).astype(jnp.float32)
        acc = acc + wtb[:, h:h + 1] * rb
    row = i * BT + jax.lax.broadcasted_iota(jnp.int32, (BT, T), 0)
    col = jax.lax.broadcasted_iota(jnp.int32, (BT, T), 1)
    out_ref[...] = jnp.where(row >= col, acc, NEG)


def _keyify(f):
    k = jax.lax.bitcast_convert_type(f, jnp.int32)
    return jnp.where(k >= 0, k, k ^ jnp.int32(0x7FFFFFFF))


def _thresh_kernel(isc_ref, tau_ref, key_ref):
    key_ref[...] = _keyify(isc_ref[...])

    def body(b, tau):
        bit = jnp.left_shift(jnp.int32(1), 30 - b)
        cand = tau + bit
        cnt = jnp.sum((key_ref[...] >= cand).astype(jnp.int32), axis=1,
                      keepdims=True)
        return jnp.where(cnt >= TOPK, cand, tau)

    cnt_pos = jnp.sum((key_ref[...] >= 0).astype(jnp.int32), axis=1,
                      keepdims=True)
    tau0 = jnp.where(cnt_pos >= TOPK, jnp.int32(0), jnp.int32(-2147483648))
    tau = jax.lax.fori_loop(0, 31, body, tau0)
    kb = jnp.where(tau >= 0, tau, tau ^ jnp.int32(0x7FFFFFFF))
    tau_ref[...] = jax.lax.bitcast_convert_type(kb, jnp.float32)


def _attn_kernel(qn_ref, qpe_ref, kn_ref, kpe_ref, v_ref, isc_ref, tau_ref,
                 o_ref, b_scr, m_scr, l_scr, acc_scr):
    i = pl.program_id(0)
    h = pl.program_id(1)
    j = pl.program_id(2)
    scaling = DQK ** -0.5

    @pl.when(jnp.logical_and(h == 0, j == 0))
    def _bias():
        row = i * BT + jax.lax.broadcasted_iota(jnp.int32, (BT, T), 0)
        col = jax.lax.broadcasted_iota(jnp.int32, (BT, T), 1)
        sel = jnp.logical_and(isc_ref[...] >= tau_ref[...], row >= col)
        b_scr[...] = jnp.where(sel, 0.0, NEG)

    @pl.when(j == 0)
    def _init():
        m_scr[...] = jnp.full((BT, 1), NEG, jnp.float32)
        l_scr[...] = jnp.zeros((BT, 1), jnp.float32)
        acc_scr[...] = jnp.zeros((BT, DV), jnp.float32)

    @pl.when(j <= i)
    def _block():
        off = pl.multiple_of(j * BT, BT)
        sj = (_dot(qn_ref[0], kn_ref[...], trans_b=True)
              + _dot(qpe_ref[0], kpe_ref[...], trans_b=True))
        sj = sj * scaling + b_scr[:, pl.ds(off, BT)]
        m_old = m_scr[...]
        m_new = jnp.maximum(m_old, jnp.max(sj, axis=1, keepdims=True))
        corr = jnp.exp(m_old - m_new)
        pj = jnp.exp(sj - m_new)
        m_scr[...] = m_new
        l_scr[...] = l_scr[...] * corr + jnp.sum(pj, axis=1, keepdims=True)
        acc_scr[...] = (acc_scr[...] * corr
                        + _dot(pj.astype(jnp.bfloat16), v_ref[...]))

    @pl.when(j == i)
    def _fin():
        o_ref[...] = acc_scr[...] / l_scr[...]


def _oproj_kernel(o_ref, wo_ref, out_ref):
    out_ref[...] = _dot(o_ref[...], wo_ref[...])


def _pipeline(hidden_states, q_c, kv_c, k_pe, positions, q_a_ln_w, W_qb,
              kv_a_ln_w, W_kvb, W_o, W_iqb, W_ik, ik_ln_w, ik_ln_b, W_w,
              return_parts=False):
    n = T

    posf = positions.astype(jnp.float32)
    half = DR // 2
    inv = jnp.asarray(
        1.0 / (10000.0 ** (np.arange(half, dtype=np.float32) / half)),
        dtype=jnp.float32)
    f = posf[:, None] * inv[None, :]
    cos, sin = jnp.cos(f), jnp.sin(f)
    sign = jnp.tile(jnp.array([-1.0, 1.0], jnp.float32), (half,))
    c_il = jnp.repeat(cos, 2, axis=1)
    s_il = jnp.repeat(sin, 2, axis=1) * sign[None, :]
    ones64 = jnp.ones((n, 64), jnp.float32)
    zeros64 = jnp.zeros((n, 64), jnp.float32)
    c_nx = jnp.concatenate([ones64, cos, cos], axis=1)
    s_nx = jnp.concatenate([zeros64, -sin, sin], axis=1)

    fspec = lambda shape, imap: pl.BlockSpec(shape, imap)

    q_nope, q_pe, q_i = pl.pallas_call(
        _proj_q_kernel,
        grid=(NB,),
        in_specs=[
            fspec((BT, RQ), lambda i: (i, 0)),
            fspec((RQ, H * DQK), lambda i: (0, 0)),
            fspec((RQ, HI * DI), lambda i: (0, 0)),
            fspec((1, RQ), lambda i: (0, 0)),
            fspec((BT, DR), lambda i: (i, 0)),
            fspec((BT, DR), lambda i: (i, 0)),
            fspec((BT, DI), lambda i: (i, 0)),
            fspec((BT, DI), lambda i: (i, 0)),
        ],
        out_specs=[
            fspec((H, BT, DN), lambda i: (0, i, 0)),
            fspec((H, BT, DR), lambda i: (0, i, 0)),
            fspec((HI, BT, DI), lambda i: (0, i, 0)),
        ],
        out_shape=[
            jax.ShapeDtypeStruct((H, n, DN), jnp.bfloat16),
            jax.ShapeDtypeStruct((H, n, DR), jnp.bfloat16),
            jax.ShapeDtypeStruct((HI, n, DI), jnp.bfloat16),
        ],
    )(q_c, W_qb, W_iqb, q_a_ln_w.reshape(1, RQ), c_il, s_il, c_nx, s_nx)

    k_nope, v, k_pe_r, k_i, w_t = pl.pallas_call(
        _proj_kv_kernel,
        grid=(NB,),
        in_specs=[
            fspec((BT, RKV), lambda i: (i, 0)),
            fspec((BT, HID), lambda i: (i, 0)),
            fspec((BT, DR), lambda i: (i, 0)),
            fspec((RKV, H * (DN + DV)), lambda i: (0, 0)),
            fspec((HID, DI), lambda i: (0, 0)),
            fspec((HID, HI), lambda i: (0, 0)),
            fspec((1, RKV), lambda i: (0, 0)),
            fspec((1, DI), lambda i: (0, 0)),
            fspec((1, DI), lambda i: (0, 0)),
            fspec((BT, DR), lambda i: (i, 0)),
            fspec((BT, DR), lambda i: (i, 0)),
            fspec((BT, DI), lambda i: (i, 0)),
            fspec((BT, DI), lambda i: (i, 0)),
        ],
        out_specs=[
            fspec((BT, H * DN), lambda i: (i, 0)),
            fspec((BT, H * DV), lambda i: (i, 0)),
            fspec((BT, DR), lambda i: (i, 0)),
            fspec((BT, DI), lambda i: (i, 0)),
            fspec((BT, HI), lambda i: (i, 0)),
        ],
        out_shape=[
            jax.ShapeDtypeStruct((n, H * DN), jnp.bfloat16),
            jax.ShapeDtypeStruct((n, H * DV), jnp.bfloat16),
            jax.ShapeDtypeStruct((n, DR), jnp.bfloat16),
            jax.ShapeDtypeStruct((n, DI), jnp.bfloat16),
            jax.ShapeDtypeStruct((n, HI), jnp.float32),
        ],
    )(kv_c, hidden_states, k_pe, W_kvb, W_ik, W_w,
      kv_a_ln_w.reshape(1, RKV), ik_ln_w.reshape(1, DI),
      ik_ln_b.reshape(1, DI), c_il, s_il, c_nx, s_nx)

    iscore = pl.pallas_call(
        _iscore_kernel,
        grid=(NB,),
        in_specs=[
            fspec((HI, BT, DI), lambda i: (0, i, 0)),
            fspec((n, DI), lambda i: (0, 0)),
            fspec((BT, HI), lambda i: (i, 0)),
        ],
        out_specs=fspec((BT, n), lambda i: (i, 0)),
        out_shape=jax.ShapeDtypeStruct((n, n), jnp.float32),
    )(q_i, k_i, w_t)

    tau = pl.pallas_call(
        _thresh_kernel,
        grid=(NB,),
        in_specs=[fspec((BT, n), lambda i: (i, 0))],
        out_specs=fspec((BT, 1), lambda i: (i, 0)),
        out_shape=jax.ShapeDtypeStruct((n, 1), jnp.float32),
        scratch_shapes=[pltpu.VMEM((BT, n), jnp.int32)],
    )(iscore)

    if return_parts:
        return q_i, k_i, w_t, iscore, tau

    o_heads = pl.pallas_call(
        _attn_kernel,
        grid=(NB, H, NB),
        in_specs=[
            pl.BlockSpec((1, BT, DN), lambda i, h, j: (h, i, 0)),
            pl.BlockSpec((1, BT, DR), lambda i, h, j: (h, i, 0)),
            fspec((BT, DN), lambda i, h, j: (j, h)),
            fspec((BT, DR), lambda i, h, j: (j, 0)),
            fspec((BT, DV), lambda i, h, j: (j, h)),
            fspec((BT, n), lambda i, h, j: (i, 0)),
            fspec((BT, 1), lambda i, h, j: (i, 0)),
        ],
        out_specs=fspec((BT, DV), lambda i, h, j: (i, h)),
        out_shape=jax.ShapeDtypeStruct((n, H * DV), jnp.float32),
        scratch_shapes=[pltpu.VMEM((BT, T), jnp.float32),
                        pltpu.VMEM((BT, 1), jnp.float32),
                        pltpu.VMEM((BT, 1), jnp.float32),
                        pltpu.VMEM((BT, DV), jnp.float32)],
        compiler_params=pltpu.CompilerParams(
            dimension_semantics=("arbitrary", "arbitrary", "arbitrary")),
    )(q_nope, q_pe, k_nope, k_pe_r, v, iscore, tau)

    out = pl.pallas_call(
        _oproj_kernel,
        grid=(NB,),
        in_specs=[
            fspec((BT, H * DV), lambda i: (i, 0)),
            fspec((H * DV, HID), lambda i: (0, 0)),
        ],
        out_specs=fspec((BT, HID), lambda i: (i, 0)),
        out_shape=jax.ShapeDtypeStruct((n, HID), jnp.float32),
    )(o_heads, W_o)

    return out


def kernel(*args):
    return _pipeline(*args)


def kernel_parts(*args):
    return _pipeline(*args, return_parts=True)


# speedup vs baseline: 1.6837x; 1.6837x over previous
import jax
import jax.numpy as jnp
import numpy as np
from jax.experimental import pallas as pl
from jax.experimental.pallas import tpu as pltpu

T = 2048
HID = 2048
H = 16
DN = 128
DR = 64
DQK = DN + DR
DV = 128
RQ = 1536
RKV = 512
HI = 8
DI = 128
TOPK = 512
EPS = 1e-6
NEG = -1e9

BT = 256
NB = T // BT


def _dot(a, b, trans_b=False):
    dn = (((1,), (1,)), ((), ())) if trans_b else (((1,), (0,)), ((), ()))
    return jax.lax.dot_general(a, b, dn,
                               preferred_element_type=jnp.float32)


def _roll_lanes(x, shift):
    if shift > 0:
        return jnp.concatenate([x[:, -shift:], x[:, :-shift]], axis=1)
    k = -shift
    return jnp.concatenate([x[:, k:], x[:, :k]], axis=1)


def _rope_interleaved(x, c, s):
    lane = jax.lax.broadcasted_iota(jnp.int32, x.shape, 1)
    swap = jnp.where(lane % 2 == 0, _roll_lanes(x, -1), _roll_lanes(x, 1))
    return x * c + swap * s


def _rope_neox128(x, c, s):
    lane = jax.lax.broadcasted_iota(jnp.int32, x.shape, 1)
    swap = jnp.where(lane < 96, _roll_lanes(x, -32), _roll_lanes(x, 32))
    return x * c + swap * s


def _proj_q_kernel(qc_ref, wq_ref, wiq_ref, gw_ref, cil_ref, sil_ref,
                   cnx_ref, snx_ref, qn_ref, qpe_ref, qi_ref):
    x = qc_ref[...]
    xn = x * jax.lax.rsqrt(jnp.mean(x * x, axis=-1, keepdims=True) + EPS)
    xn = xn * gw_ref[...]
    q = _dot(xn, wq_ref[...])
    cil, sil = cil_ref[...], sil_ref[...]
    for h in range(H):
        base = h * DQK
        qn_ref[h] = q[:, base:base + DN].astype(jnp.bfloat16)
        pe = q[:, base + DN:base + DQK]
        qpe_ref[h] = _rope_interleaved(pe, cil, sil).astype(jnp.bfloat16)
    qi = _dot(xn, wiq_ref[...])
    cnx, snx = cnx_ref[...], snx_ref[...]
    for h in range(HI):
        g = qi[:, h * DI:(h + 1) * DI]
        qi_ref[h] = _rope_neox128(g, cnx, snx).astype(jnp.bfloat16)


def _proj_kv_kernel(kvc_ref, hid_ref, kpe_ref, wkv_ref, wik_ref, ww_ref,
                    gkv_ref, ilw_ref, ilb_ref, cil_ref, sil_ref,
                    cnx_ref, snx_ref, kn_ref, v_ref, kper_ref, ki_ref, wt_ref):
    x = kvc_ref[...]
    xn = x * jax.lax.rsqrt(jnp.mean(x * x, axis=-1, keepdims=True) + EPS)
    xn = xn * gkv_ref[...]
    kvb = _dot(xn, wkv_ref[...])
    kn_ref[...] = jnp.concatenate(
        [kvb[:, h * (DN + DV):h * (DN + DV) + DN] for h in range(H)],
        axis=1).astype(jnp.bfloat16)
    v_ref[...] = jnp.concatenate(
        [kvb[:, h * (DN + DV) + DN:(h + 1) * (DN + DV)] for h in range(H)],
        axis=1).astype(jnp.bfloat16)

    kper_ref[...] = _rope_interleaved(
        kpe_ref[...], cil_ref[...], sil_ref[...]).astype(jnp.bfloat16)

    hdd = hid_ref[...]
    ki0 = _dot(hdd, wik_ref[...])
    m = jnp.mean(ki0, axis=-1, keepdims=True)
    d = ki0 - m
    var = jnp.mean(d * d, axis=-1, keepdims=True)
    ki = d * jax.lax.rsqrt(var + 1e-6) * ilw_ref[...] + ilb_ref[...]
    ki_ref[...] = _rope_neox128(ki, cnx_ref[...], snx_ref[...]).astype(
        jnp.bfloat16)

    wt_ref[...] = _dot(hdd, ww_ref[...]) * (HI ** -0.5)


def _iscore_kernel(qi_ref, ki_ref, wt_ref, out_ref):
    i = pl.program_id(0)
    ki = ki_ref[...]
    scale = DI ** -0.5
    wtb = wt_ref[...].astype(jnp.bfloat16).astype(jnp.float32)
    acc = jnp.zeros((BT, T), jnp.float32)
    for h in range(HI):
        sh = _dot(qi_ref[h], ki, trans_b=True) * scale
        rb = jnp.maximum(sh, 0.0).astype(jnp.bfloat16).astype(jnp.float32)
        acc = acc + wtb[:, h:h + 1] * rb
    row = i * BT + jax.lax.broadcasted_iota(jnp.int32, (BT, T), 0)
    col = jax.lax.broadcasted_iota(jnp.int32, (BT, T), 1)
    out_ref[...] = jnp.where(row >= col, acc, NEG)


def _keyify(f):
    k = jax.lax.bitcast_convert_type(f, jnp.int32)
    return jnp.where(k >= 0, k, k ^ jnp.int32(0x7FFFFFFF))


def _thresh_kernel(isc_ref, tau_ref, key_ref):
    key_ref[...] = _keyify(isc_ref[...])

    def body(b, tau):
        bit = jnp.left_shift(jnp.int32(1), 30 - b)
        cand = tau + bit
        cnt = jnp.sum((key_ref[...] >= cand).astype(jnp.int32), axis=1,
                      keepdims=True)
        return jnp.where(cnt >= TOPK, cand, tau)

    cnt_pos = jnp.sum((key_ref[...] >= 0).astype(jnp.int32), axis=1,
                      keepdims=True)
    tau0 = jnp.where(cnt_pos >= TOPK, jnp.int32(0), jnp.int32(-2147483648))
    tau = jax.lax.fori_loop(0, 31, body, tau0)
    kb = jnp.where(tau >= 0, tau, tau ^ jnp.int32(0x7FFFFFFF))
    tau_ref[...] = jax.lax.bitcast_convert_type(kb, jnp.float32)


def _attn_kernel(qn_ref, qpe_ref, kn_ref, kpe_ref, v_ref, isc_ref, tau_ref,
                 o_ref, s_scr, b_scr):
    i = pl.program_id(0)
    h = pl.program_id(1)
    scaling = DQK ** -0.5

    @pl.when(h == 0)
    def _bias():
        row = i * BT + jax.lax.broadcasted_iota(jnp.int32, (BT, T), 0)
        col = jax.lax.broadcasted_iota(jnp.int32, (BT, T), 1)
        sel = jnp.logical_and(isc_ref[...] >= tau_ref[...], row >= col)
        b_scr[...] = jnp.where(sel, 0.0, NEG)

    qn = qn_ref[0]
    qpe = qpe_ref[0]

    def j1(j, m):
        off = pl.multiple_of(j * BT, BT)
        sj = (_dot(qn, kn_ref[pl.ds(off, BT), :], trans_b=True)
              + _dot(qpe, kpe_ref[pl.ds(off, BT), :], trans_b=True))
        sj = sj * scaling + b_scr[:, pl.ds(off, BT)]
        s_scr[:, pl.ds(off, BT)] = sj
        return jnp.maximum(m, jnp.max(sj, axis=1, keepdims=True))

    m = jax.lax.fori_loop(0, i + 1, j1,
                          jnp.full((BT, 1), NEG, jnp.float32))

    def j2(j, carry):
        l, acc = carry
        off = pl.multiple_of(j * BT, BT)
        pj = jnp.exp(s_scr[:, pl.ds(off, BT)] - m)
        l = l + jnp.sum(pj, axis=1, keepdims=True)
        acc = acc + _dot(pj.astype(jnp.bfloat16), v_ref[pl.ds(off, BT), :])
        return l, acc

    l, acc = jax.lax.fori_loop(
        0, i + 1, j2,
        (jnp.zeros((BT, 1), jnp.float32), jnp.zeros((BT, DV), jnp.float32)))
    o_ref[...] = acc / l


def _oproj_kernel(o_ref, wo_ref, out_ref):
    out_ref[...] = _dot(o_ref[...], wo_ref[...])


def kernel(hidden_states, q_c, kv_c, k_pe, positions, q_a_ln_w, W_qb,
           kv_a_ln_w, W_kvb, W_o, W_iqb, W_ik, ik_ln_w, ik_ln_b, W_w):
    n = T

    posf = positions.astype(jnp.float32)
    half = DR // 2
    inv = jnp.asarray(
        1.0 / (10000.0 ** (np.arange(half, dtype=np.float32) / half)),
        dtype=jnp.float32)
    f = posf[:, None] * inv[None, :]
    cos, sin = jnp.cos(f), jnp.sin(f)
    sign = jnp.tile(jnp.array([-1.0, 1.0], jnp.float32), (half,))
    c_il = jnp.repeat(cos, 2, axis=1)
    s_il = jnp.repeat(sin, 2, axis=1) * sign[None, :]
    ones64 = jnp.ones((n, 64), jnp.float32)
    zeros64 = jnp.zeros((n, 64), jnp.float32)
    c_nx = jnp.concatenate([ones64, cos, cos], axis=1)
    s_nx = jnp.concatenate([zeros64, -sin, sin], axis=1)

    fspec = lambda shape, imap: pl.BlockSpec(shape, imap)

    q_nope, q_pe, q_i = pl.pallas_call(
        _proj_q_kernel,
        grid=(NB,),
        in_specs=[
            fspec((BT, RQ), lambda i: (i, 0)),
            fspec((RQ, H * DQK), lambda i: (0, 0)),
            fspec((RQ, HI * DI), lambda i: (0, 0)),
            fspec((1, RQ), lambda i: (0, 0)),
            fspec((BT, DR), lambda i: (i, 0)),
            fspec((BT, DR), lambda i: (i, 0)),
            fspec((BT, DI), lambda i: (i, 0)),
            fspec((BT, DI), lambda i: (i, 0)),
        ],
        out_specs=[
            fspec((H, BT, DN), lambda i: (0, i, 0)),
            fspec((H, BT, DR), lambda i: (0, i, 0)),
            fspec((HI, BT, DI), lambda i: (0, i, 0)),
        ],
        out_shape=[
            jax.ShapeDtypeStruct((H, n, DN), jnp.bfloat16),
            jax.ShapeDtypeStruct((H, n, DR), jnp.bfloat16),
            jax.ShapeDtypeStruct((HI, n, DI), jnp.bfloat16),
        ],
    )(q_c, W_qb, W_iqb, q_a_ln_w.reshape(1, RQ), c_il, s_il, c_nx, s_nx)

    k_nope, v, k_pe_r, k_i, w_t = pl.pallas_call(
        _proj_kv_kernel,
        grid=(NB,),
        in_specs=[
            fspec((BT, RKV), lambda i: (i, 0)),
            fspec((BT, HID), lambda i: (i, 0)),
            fspec((BT, DR), lambda i: (i, 0)),
            fspec((RKV, H * (DN + DV)), lambda i: (0, 0)),
            fspec((HID, DI), lambda i: (0, 0)),
            fspec((HID, HI), lambda i: (0, 0)),
            fspec((1, RKV), lambda i: (0, 0)),
            fspec((1, DI), lambda i: (0, 0)),
            fspec((1, DI), lambda i: (0, 0)),
            fspec((BT, DR), lambda i: (i, 0)),
            fspec((BT, DR), lambda i: (i, 0)),
            fspec((BT, DI), lambda i: (i, 0)),
            fspec((BT, DI), lambda i: (i, 0)),
        ],
        out_specs=[
            fspec((BT, H * DN), lambda i: (i, 0)),
            fspec((BT, H * DV), lambda i: (i, 0)),
            fspec((BT, DR), lambda i: (i, 0)),
            fspec((BT, DI), lambda i: (i, 0)),
            fspec((BT, HI), lambda i: (i, 0)),
        ],
        out_shape=[
            jax.ShapeDtypeStruct((n, H * DN), jnp.bfloat16),
            jax.ShapeDtypeStruct((n, H * DV), jnp.bfloat16),
            jax.ShapeDtypeStruct((n, DR), jnp.bfloat16),
            jax.ShapeDtypeStruct((n, DI), jnp.bfloat16),
            jax.ShapeDtypeStruct((n, HI), jnp.float32),
        ],
    )(kv_c, hidden_states, k_pe, W_kvb, W_ik, W_w,
      kv_a_ln_w.reshape(1, RKV), ik_ln_w.reshape(1, DI),
      ik_ln_b.reshape(1, DI), c_il, s_il, c_nx, s_nx)

    iscore = pl.pallas_call(
        _iscore_kernel,
        grid=(NB,),
        in_specs=[
            fspec((HI, BT, DI), lambda i: (0, i, 0)),
            fspec((n, DI), lambda i: (0, 0)),
            fspec((BT, HI), lambda i: (i, 0)),
        ],
        out_specs=fspec((BT, n), lambda i: (i, 0)),
        out_shape=jax.ShapeDtypeStruct((n, n), jnp.float32),
    )(q_i, k_i, w_t)

    tau = pl.pallas_call(
        _thresh_kernel,
        grid=(NB,),
        in_specs=[fspec((BT, n), lambda i: (i, 0))],
        out_specs=fspec((BT, 1), lambda i: (i, 0)),
        out_shape=jax.ShapeDtypeStruct((n, 1), jnp.float32),
        scratch_shapes=[pltpu.VMEM((BT, n), jnp.int32)],
    )(iscore)

    o_heads = pl.pallas_call(
        _attn_kernel,
        grid=(NB, H),
        in_specs=[
            pl.BlockSpec((1, BT, DN), lambda i, h: (h, i, 0)),
            pl.BlockSpec((1, BT, DR), lambda i, h: (h, i, 0)),
            fspec((n, DN), lambda i, h: (0, h)),
            fspec((n, DR), lambda i, h: (0, 0)),
            fspec((n, DV), lambda i, h: (0, h)),
            fspec((BT, n), lambda i, h: (i, 0)),
            fspec((BT, 1), lambda i, h: (i, 0)),
        ],
        out_specs=fspec((BT, DV), lambda i, h: (i, h)),
        out_shape=jax.ShapeDtypeStruct((n, H * DV), jnp.float32),
        scratch_shapes=[pltpu.VMEM((BT, T), jnp.float32),
                        pltpu.VMEM((BT, T), jnp.float32)],
        compiler_params=pltpu.CompilerParams(
            dimension_semantics=("arbitrary", "arbitrary")),
    )(q_nope, q_pe, k_nope, k_pe_r, v, iscore, tau)

    out = pl.pallas_call(
        _oproj_kernel,
        grid=(NB,),
        in_specs=[
            fspec((BT, H * DV), lambda i: (i, 0)),
            fspec((H * DV, HID), lambda i: (0, 0)),
        ],
        out_specs=fspec((BT, HID), lambda i: (i, 0)),
        out_shape=jax.ShapeDtypeStruct((n, HID), jnp.float32),
    )(o_heads, W_o)

    return out
